# baseline (device time: 59479 ns/iter reference)
import jax
import jax.numpy as jnp
from jax import lax
from jax.experimental import pallas as pl
from jax.experimental.pallas import tpu as pltpu

N_DEV = 4


def kernel(x, w_mat, scale_x, scale_w):
    fp8 = jnp.float8_e4m3fn
    x = x.astype(fp8)
    w_mat = w_mat.astype(fp8)

    m_total, k_shard = x.shape
    k_total, n = w_mat.shape
    m_blk = m_total // N_DEV

    def body(x_ref, w_ref, sx_ref, sw_ref, out_ref,
             comm_ref, send_sems, recv_sems):
        my_pos = lax.axis_index("i")

        barrier_sem = pltpu.get_barrier_semaphore()
        for d in range(1, N_DEV):
            peer = (my_pos + d) % N_DEV
            pl.semaphore_signal(
                barrier_sem, inc=1,
                device_id=(peer,), device_id_type=pl.DeviceIdType.MESH,
            )
        pl.semaphore_wait(barrier_sem, N_DEV - 1)

        for s in range(N_DEV):
            @pl.when(my_pos == s)
            def _(s=s):
                comm_ref[s] = x_ref[s * m_blk:(s + 1) * m_blk, :]

        sends = []
        for d in range(1, N_DEV):
            peer = (my_pos + d) % N_DEV
            rdma = pltpu.make_async_remote_copy(
                src_ref=x_ref.at[pl.ds(peer * m_blk, m_blk), :],
                dst_ref=comm_ref.at[my_pos],
                send_sem=send_sems.at[d - 1],
                recv_sem=recv_sems.at[my_pos],
                device_id=(peer,),
                device_id_type=pl.DeviceIdType.MESH,
            )
            rdma.start()
            sends.append(rdma)

        for d in range(1, N_DEV):
            src_pos = (my_pos + d) % N_DEV
            recv = pltpu.make_async_remote_copy(
                src_ref=x_ref.at[pl.ds(0, m_blk), :],
                dst_ref=comm_ref.at[src_pos],
                send_sem=send_sems.at[N_DEV - 1],
                recv_sem=recv_sems.at[src_pos],
                device_id=(src_pos,),
                device_id_type=pl.DeviceIdType.MESH,
            )
            recv.wait_recv()

        k_blk = k_shard
        acc = jnp.zeros((m_blk, n), jnp.float32)
        for s in range(N_DEV):
            acc += lax.dot_general(
                comm_ref[s], w_ref[s * k_blk:(s + 1) * k_blk, :],
                dimension_numbers=(((1,), (0,)), ((), ())),
                preferred_element_type=jnp.float32,
            )
        out_ref[...] = acc * (sx_ref[0] * sw_ref[0])

        for rdma in sends:
            rdma.wait_send()

    return pl.pallas_call(
        body,
        out_shape=jax.ShapeDtypeStruct((m_blk, n), jnp.float32),
        in_specs=[
            pl.BlockSpec(memory_space=pltpu.VMEM),
            pl.BlockSpec(memory_space=pltpu.VMEM),
            pl.BlockSpec(memory_space=pltpu.SMEM),
            pl.BlockSpec(memory_space=pltpu.SMEM),
        ],
        out_specs=pl.BlockSpec(memory_space=pltpu.VMEM),
        scratch_shapes=[
            pltpu.VMEM((N_DEV, m_blk, k_shard), fp8),
            pltpu.SemaphoreType.DMA((N_DEV,)),
            pltpu.SemaphoreType.DMA((N_DEV,)),
        ],
        compiler_params=pltpu.CompilerParams(collective_id=0),
    )(x, w_mat, scale_x, scale_w)


# device time: 50352 ns/iter; 1.1813x vs baseline; 1.1813x over previous
import jax
import jax.numpy as jnp
from jax import lax
from jax.experimental import pallas as pl
from jax.experimental.pallas import tpu as pltpu

N_DEV = 4


def kernel(x, w_mat, scale_x, scale_w):
    fp8 = jnp.float8_e4m3fn
    m_total, k_blk = x.shape
    k_total, n = w_mat.shape
    m_blk = m_total // N_DEV
    assert k_total == N_DEV * k_blk

    def body(x_hbm, w_hbm, sx_ref, sw_ref, out_ref,
             xf32_ref, xq_ref, comm_ref, wstage_ref, wq_ref,
             send_sems, recv_sems, xdma_sem, wdma_sems):
        my_pos = lax.axis_index("i")

        def w_copy(s):
            return pltpu.make_async_copy(
                w_hbm.at[pl.ds(s * k_blk, k_blk), :],
                wstage_ref.at[s % 2],
                wdma_sems.at[s % 2],
            )

        xcopy = pltpu.make_async_copy(x_hbm, xf32_ref, xdma_sem)
        xcopy.start()

        barrier_sem = pltpu.get_barrier_semaphore()
        for d in range(1, N_DEV):
            peer = (my_pos + d) % N_DEV
            pl.semaphore_signal(
                barrier_sem, inc=1,
                device_id=(peer,), device_id_type=pl.DeviceIdType.MESH,
            )
        pl.semaphore_wait(barrier_sem, N_DEV - 1)

        w_copy(0).start()
        w_copy(1).start()

        xcopy.wait()

        for s in range(N_DEV):
            xq_ref[s] = xf32_ref[s * m_blk:(s + 1) * m_blk, :].astype(fp8)

            @pl.when(my_pos != s)
            def _(s=s):
                rdma = pltpu.make_async_remote_copy(
                    src_ref=xq_ref.at[s],
                    dst_ref=comm_ref.at[my_pos],
                    send_sem=send_sems.at[s],
                    recv_sem=recv_sems.at[my_pos],
                    device_id=(s,),
                    device_id_type=pl.DeviceIdType.MESH,
                )
                rdma.start()

            @pl.when(my_pos == s)
            def _(s=s):
                comm_ref[s] = xq_ref[s]

        scale = sx_ref[0] * sw_ref[0]
        for s in range(N_DEV):
            slot = s % 2
            w_copy(s).wait()
            wq_ref[slot] = wstage_ref[slot].astype(fp8)
            if s + 2 < N_DEV:
                w_copy(s + 2).start()

            @pl.when(my_pos != s)
            def _(s=s):
                recv = pltpu.make_async_remote_copy(
                    src_ref=xq_ref.at[s],
                    dst_ref=comm_ref.at[s],
                    send_sem=send_sems.at[s],
                    recv_sem=recv_sems.at[s],
                    device_id=(s,),
                    device_id_type=pl.DeviceIdType.MESH,
                )
                recv.wait_recv()

            contrib = lax.dot_general(
                comm_ref[s], wq_ref[slot],
                dimension_numbers=(((1,), (0,)), ((), ())),
                preferred_element_type=jnp.float32,
            )
            if s == 0:
                out_ref[...] = contrib
            elif s == N_DEV - 1:
                out_ref[...] = (out_ref[...] + contrib) * scale
            else:
                out_ref[...] = out_ref[...] + contrib

        for s in range(N_DEV):
            @pl.when(my_pos != s)
            def _(s=s):
                send = pltpu.make_async_remote_copy(
                    src_ref=xq_ref.at[s],
                    dst_ref=comm_ref.at[my_pos],
                    send_sem=send_sems.at[s],
                    recv_sem=recv_sems.at[my_pos],
                    device_id=(s,),
                    device_id_type=pl.DeviceIdType.MESH,
                )
                send.wait_send()

    return pl.pallas_call(
        body,
        out_shape=jax.ShapeDtypeStruct((m_blk, n), jnp.float32),
        in_specs=[
            pl.BlockSpec(memory_space=pl.ANY),
            pl.BlockSpec(memory_space=pl.ANY),
            pl.BlockSpec(memory_space=pltpu.SMEM),
            pl.BlockSpec(memory_space=pltpu.SMEM),
        ],
        out_specs=pl.BlockSpec(memory_space=pltpu.VMEM),
        scratch_shapes=[
            pltpu.VMEM((m_total, k_blk), jnp.float32),
            pltpu.VMEM((N_DEV, m_blk, k_blk), fp8),
            pltpu.VMEM((N_DEV, m_blk, k_blk), fp8),
            pltpu.VMEM((2, k_blk, n), jnp.float32),
            pltpu.VMEM((2, k_blk, n), fp8),
            pltpu.SemaphoreType.DMA((N_DEV,)),
            pltpu.SemaphoreType.DMA((N_DEV,)),
            pltpu.SemaphoreType.DMA,
            pltpu.SemaphoreType.DMA((2,)),
        ],
        compiler_params=pltpu.CompilerParams(
            collective_id=0,
            vmem_limit_bytes=60 * 1024 * 1024,
        ),
    )(x, w_mat, scale_x, scale_w)


# device time: 42424 ns/iter; 1.4020x vs baseline; 1.1869x over previous
import jax
import jax.numpy as jnp
from jax import lax
from jax.experimental import pallas as pl
from jax.experimental.pallas import tpu as pltpu

N_DEV = 4


def kernel(x, w_mat, scale_x, scale_w):
    fp8 = jnp.float8_e4m3fn
    m_total, k_blk = x.shape
    k_total, n = w_mat.shape
    m_blk = m_total // N_DEV
    assert k_total == N_DEV * k_blk

    def body(x_hbm, w_hbm, sx_ref, sw_ref, out_hbm,
             xstage_ref, xq_ref, comm_ref, wstage_ref, wq_ref, acc_ref,
             send_sems, recv_sems, xdma_sems, wdma_sems, odma_sems):
        my_pos = lax.axis_index("i")
        hm = m_blk // 2

        class _Multi:
            def __init__(self, copies):
                self.copies = copies

            def start(self):
                for c in self.copies:
                    c.start()

            def wait(self):
                for c in self.copies:
                    c.wait()

        def x_copy(s, slot, nch=4):
            rows = m_blk // nch
            return _Multi([
                pltpu.make_async_copy(
                    x_hbm.at[pl.ds(s * m_blk + c * rows, rows), :],
                    xstage_ref.at[slot, pl.ds(c * rows, rows), :],
                    xdma_sems.at[slot, c],
                )
                for c in range(nch)
            ])

        def w_copy(s, slot, nch=2):
            rows = k_blk // nch
            return _Multi([
                pltpu.make_async_copy(
                    w_hbm.at[pl.ds(s * k_blk + c * rows, rows), :],
                    wstage_ref.at[slot, pl.ds(c * rows, rows), :],
                    wdma_sems.at[slot, c],
                )
                for c in range(nch)
            ])

        for p in range(N_DEV):
            @pl.when(my_pos == p)
            def _(p=p):
                xorder = [(p + d) % N_DEV for d in (1, 2, 3)] + [p]
                x_copy(xorder[0], 0).start()
                x_copy(xorder[1], 1).start()

        barrier_sem = pltpu.get_barrier_semaphore()
        for d in range(1, N_DEV):
            peer = (my_pos + d) % N_DEV
            pl.semaphore_signal(
                barrier_sem, inc=1,
                device_id=(peer,), device_id_type=pl.DeviceIdType.MESH,
            )
        pl.semaphore_wait(barrier_sem, N_DEV - 1)

        for p in range(N_DEV):
            @pl.when(my_pos == p)
            def _(p=p):
                xorder = [(p + d) % N_DEV for d in (1, 2, 3)] + [p]
                for i, s in enumerate(xorder):
                    x_copy(s, i % 2).wait()
                    xq_ref[s] = (
                        xstage_ref[i % 2].astype(jnp.bfloat16).astype(fp8)
                    )
                    if i + 2 < N_DEV:
                        x_copy(xorder[i + 2], i % 2).start()
                    if s != p:
                        for h in range(2):
                            rdma = pltpu.make_async_remote_copy(
                                src_ref=xq_ref.at[s, pl.ds(h * hm, hm), :],
                                dst_ref=comm_ref.at[p, pl.ds(h * hm, hm), :],
                                send_sem=send_sems.at[s, h],
                                recv_sem=recv_sems.at[p, h],
                                device_id=(s,),
                                device_id_type=pl.DeviceIdType.MESH,
                            )
                            rdma.start()

        def dot(a_ref, s):
            return lax.dot_general(
                a_ref[s], wq_ref[s],
                dimension_numbers=(((1,), (0,)), ((), ())),
                preferred_element_type=jnp.float32,
            )

        for p in range(N_DEV):
            @pl.when(my_pos == p)
            def _(p=p):
                worder = [(p - d) % N_DEV for d in range(N_DEV)]
                w_copy(worder[0], 0).start()
                w_copy(worder[1], 1).start()
                for i, s in enumerate(worder):
                    w_copy(s, i % 2).wait()
                    wq_ref[s] = (
                        wstage_ref[i % 2].astype(jnp.bfloat16).astype(fp8)
                    )
                    if i + 2 < N_DEV:
                        w_copy(worder[i + 2], i % 2).start()
                    if i == 0:
                        acc_ref[...] = dot(xq_ref, p)

        scale = sx_ref[0] * sw_ref[0]

        def out_copy(h):
            return pltpu.make_async_copy(
                acc_ref.at[pl.ds(h * hm, hm), :],
                out_hbm.at[pl.ds(h * hm, hm), :],
                odma_sems.at[h],
            )

        for d in (1, 2, 3):
            for p in range(N_DEV):
                @pl.when(my_pos == p)
                def _(p=p, d=d):
                    s = (p - d) % N_DEV
                    for h in range(2):
                        recv = pltpu.make_async_remote_copy(
                            src_ref=xq_ref.at[s, pl.ds(h * hm, hm), :],
                            dst_ref=comm_ref.at[s, pl.ds(h * hm, hm), :],
                            send_sem=send_sems.at[s, h],
                            recv_sem=recv_sems.at[s, h],
                            device_id=(s,),
                            device_id_type=pl.DeviceIdType.MESH,
                        )
                        recv.wait_recv()
                        contrib = lax.dot_general(
                            comm_ref[s, h * hm:(h + 1) * hm, :], wq_ref[s],
                            dimension_numbers=(((1,), (0,)), ((), ())),
                            preferred_element_type=jnp.float32,
                        )
                        if d < 3:
                            acc_ref[h * hm:(h + 1) * hm, :] = (
                                acc_ref[h * hm:(h + 1) * hm, :] + contrib
                            )
                        else:
                            acc_ref[h * hm:(h + 1) * hm, :] = (
                                acc_ref[h * hm:(h + 1) * hm, :] + contrib
                            ) * scale
                            out_copy(h).start()

        for s in range(N_DEV):
            @pl.when(my_pos != s)
            def _(s=s):
                for h in range(2):
                    send = pltpu.make_async_remote_copy(
                        src_ref=xq_ref.at[s, pl.ds(h * hm, hm), :],
                        dst_ref=comm_ref.at[s, pl.ds(h * hm, hm), :],
                        send_sem=send_sems.at[s, h],
                        recv_sem=recv_sems.at[s, h],
                        device_id=(s,),
                        device_id_type=pl.DeviceIdType.MESH,
                    )
                    send.wait_send()
        out_copy(0).wait()
        out_copy(1).wait()

    return pl.pallas_call(
        body,
        out_shape=jax.ShapeDtypeStruct((m_blk, n), jnp.float32),
        in_specs=[
            pl.BlockSpec(memory_space=pl.ANY),
            pl.BlockSpec(memory_space=pl.ANY),
            pl.BlockSpec(memory_space=pltpu.SMEM),
            pl.BlockSpec(memory_space=pltpu.SMEM),
        ],
        out_specs=pl.BlockSpec(memory_space=pl.ANY),
        scratch_shapes=[
            pltpu.VMEM((2, m_blk, k_blk), jnp.float32),
            pltpu.VMEM((N_DEV, m_blk, k_blk), fp8),
            pltpu.VMEM((N_DEV, m_blk, k_blk), fp8),
            pltpu.VMEM((2, k_blk, n), jnp.float32),
            pltpu.VMEM((N_DEV, k_blk, n), fp8),
            pltpu.VMEM((m_blk, n), jnp.float32),
            pltpu.SemaphoreType.DMA((N_DEV, 2)),
            pltpu.SemaphoreType.DMA((N_DEV, 2)),
            pltpu.SemaphoreType.DMA((2, 4)),
            pltpu.SemaphoreType.DMA((2, 2)),
            pltpu.SemaphoreType.DMA((2,)),
        ],
        compiler_params=pltpu.CompilerParams(
            collective_id=0,
            vmem_limit_bytes=60 * 1024 * 1024,
        ),
    )(x, w_mat, scale_x, scale_w)
